# Initial kernel scaffold; baseline (speedup 1.0000x reference)
import jax
import jax.numpy as jnp
from jax.experimental import pallas as pl


def kernel(x, edge_index, W1, att_src1, att_dst1, bias1, W2, att_src2, att_dst2, bias2):
    raise NotImplementedError("write your pallas kernel here")


# speedup vs baseline: 4.9843x; 4.9843x over previous
import functools

import jax
import jax.numpy as jnp
from jax import lax
from jax.experimental import pallas as pl
from jax.experimental.pallas import tpu as pltpu
from jax.experimental.pallas import tpu_sc as plsc

N_NODES = 10000
N_EDGES_RAW = 320000
E_TOT = N_EDGES_RAW + N_NODES
IN_CH = 128
HIDDEN = 128
HEADS = 4
OUT_CH = 128

NC, NS, L = 2, 16, 16
NW = NC * NS
NPT = 320
RP_LEN = NW * NPT + 16
SRC_PAD = 40


def _tc_layer1_body(x_ref, w_ref, a_ref, h_ref, a1_ref):
    h = jnp.dot(x_ref[...], w_ref[...], preferred_element_type=jnp.float32)
    h_ref[...] = h
    a1_ref[...] = jnp.dot(h, a_ref[...], preferred_element_type=jnp.float32)


def _tc_layer2_body(g_ref, w_ref, a_ref, h_ref, a2_ref):
    t = g_ref[...]
    t = jnp.where(t > 0, t, jnp.exp(t) - 1.0)
    h = jnp.dot(t, w_ref[...], preferred_element_type=jnp.float32)
    h_ref[...] = h
    a2_ref[...] = jnp.dot(h, a_ref[...], preferred_element_type=jnp.float32)


def _tc_project(body, xin, W, A, bm):
    n = xin.shape[0]
    cin = xin.shape[1]
    cout = W.shape[1]
    ca = A.shape[1]
    grid = n // bm
    return pl.pallas_call(
        body,
        grid=(grid,),
        in_specs=[
            pl.BlockSpec((bm, cin), lambda i: (i, 0)),
            pl.BlockSpec((cin, cout), lambda i: (0, 0)),
            pl.BlockSpec((cout, ca), lambda i: (0, 0)),
        ],
        out_specs=[
            pl.BlockSpec((bm, cout), lambda i: (i, 0)),
            pl.BlockSpec((bm, ca), lambda i: (i, 0)),
        ],
        out_shape=[
            jax.ShapeDtypeStruct((n, cout), jnp.float32),
            jax.ShapeDtypeStruct((n, ca), jnp.float32),
        ],
    )(xin, W, A)


def _make_sc_gat(H, C):
    HC = H * C
    SLC = C // L
    mesh = plsc.VectorSubcoreMesh(core_axis_name="c", subcore_axis_name="s")

    @functools.partial(
        pl.kernel,
        mesh=mesh,
        compiler_params=pltpu.CompilerParams(needs_layout_passes=False),
        out_type=jax.ShapeDtypeStruct((N_NODES * HC,), jnp.float32),
        scratch_types=[
            pltpu.VMEM((N_NODES * 2 * H,), jnp.float32),
            pltpu.VMEM((NPT + 16,), jnp.int32),
            pltpu.VMEM((32,), jnp.int32),
            pltpu.VMEM((L,), jnp.int32),
            pltpu.VMEM((L, HC), jnp.float32),
            pltpu.VMEM((HC,), jnp.float32),
            pltpu.VMEM((HC,), jnp.float32),
            pltpu.SemaphoreType.DMA,
        ],
    )
    def k(h_hbm, a_hbm, rp_hbm, src_hbm, bias_hbm, out_hbm,
          a_v, rp_v, win_v, idx_v, rows_v, or_v, b_v, sem):
        wid = lax.axis_index("s") * NC + lax.axis_index("c")
        n0 = pl.multiple_of(wid * NPT, NPT)
        pltpu.sync_copy(a_hbm, a_v)
        pltpu.sync_copy(rp_hbm.at[pl.ds(n0, NPT + 16)], rp_v)
        pltpu.sync_copy(bias_hbm, b_v)
        iota16 = lax.iota(jnp.int32, L)
        cnt = jnp.maximum(jnp.minimum(NPT, N_NODES - n0), 0)

        def per_dst(j, _):
            d = n0 + j
            jsplat = jnp.full((L,), 0, jnp.int32) + j
            p0 = plsc.load_gather(rp_v, [jsplat])[0]
            p1 = plsc.load_gather(rp_v, [jsplat + 1])[0]
            nch = (p1 - p0 + (L - 1)) // L
            dsplat = jnp.full((L,), 0, jnp.int32) + d * (2 * H)
            ad = [plsc.load_gather(a_v, [dsplat + (H + h)]) for h in range(H)]

            def per_chunk(kk, carry):
                accs, ms, dens = carry
                base = p0 + kk * L
                rem = p1 - base
                lanemask = iota16 < rem
                b8 = pl.multiple_of(jnp.bitwise_and(base, -8), 8)
                pltpu.sync_copy(src_hbm.at[pl.ds(b8, 32)], win_v)
                src16 = plsc.load_gather(win_v, [(base - b8) + iota16])
                src16 = jnp.where(lanemask, src16, 0)
                idx_v[...] = src16
                cp = pltpu.async_copy(h_hbm.at[idx_v], rows_v, sem)
                new_ms, new_dens, scales, ws = [], [], [], []
                src_a = src16 * (2 * H)
                for h in range(H):
                    logit = plsc.load_gather(a_v, [src_a + h]) + ad[h]
                    logit = jnp.where(logit > 0, logit, logit * 0.2)
                    logit = jnp.where(lanemask, logit, -1e30)
                    mnew = jnp.maximum(ms[h], jnp.max(logit))
                    scale = jnp.exp(ms[h] - mnew)
                    e16 = jnp.where(lanemask, jnp.exp(logit - mnew), 0.0)
                    ws.append(e16)
                    new_ms.append(mnew)
                    new_dens.append(dens[h] * scale + jnp.sum(e16))
                    scales.append(scale)
                accs = [accs[i] * scales[i // SLC] for i in range(H * SLC)]
                cp.wait()
                for e in range(L):
                    for h in range(H):
                        we = ws[h][e]
                        for s in range(SLC):
                            c0 = h * C + s * L
                            i = h * SLC + s
                            accs[i] = accs[i] + we * rows_v[e, c0:c0 + L]
                return accs, new_ms, new_dens

            zeros = jnp.zeros((L,), jnp.float32)
            init = ([zeros] * (H * SLC),
                    [jnp.full((L,), -1e30, jnp.float32)] * H,
                    [zeros] * H)
            accs, ms, dens = lax.fori_loop(0, nch, per_chunk, init)
            for h in range(H):
                inv = 1.0 / (dens[h] + 1e-16)
                for s in range(SLC):
                    c0 = h * C + s * L
                    or_v[c0:c0 + L] = accs[h * SLC + s] * inv + b_v[c0:c0 + L]
            pltpu.sync_copy(or_v, out_hbm.at[pl.ds(pl.multiple_of(d * HC, HC), HC)])
            return 0

        lax.fori_loop(0, cnt, per_dst, 0)

    return k


def kernel(x, edge_index, W1, att_src1, att_dst1, bias1, W2, att_src2, att_dst2, bias2):
    N = N_NODES
    f32 = jnp.float32
    i32 = jnp.int32

    ei = edge_index.astype(i32)
    loop = jnp.arange(N, dtype=i32)
    src0 = jnp.concatenate([ei[0], loop])
    dst0 = jnp.concatenate([ei[1], loop])
    dst_s, src_s = lax.sort((dst0, src0), num_keys=1)
    row_ptr = jnp.searchsorted(dst_s, jnp.arange(N + 1, dtype=i32),
                               side="left").astype(i32)
    rp_pad = jnp.concatenate(
        [row_ptr, jnp.full((RP_LEN - (N + 1),), E_TOT, i32)])
    src_pad = jnp.concatenate([src_s, jnp.zeros((SRC_PAD,), i32)])

    as1 = att_src1.reshape(HEADS, HIDDEN)
    ad1 = att_dst1.reshape(HEADS, HIDDEN)
    eye = jnp.eye(HEADS, dtype=f32)
    A1s = (eye[:, None, :] * as1[:, :, None]).reshape(HEADS * HIDDEN, HEADS)
    A1d = (eye[:, None, :] * ad1[:, :, None]).reshape(HEADS * HIDDEN, HEADS)
    A1 = jnp.concatenate([A1s, A1d], axis=1)
    A2 = jnp.concatenate([att_src2.reshape(OUT_CH, 1),
                          att_dst2.reshape(OUT_CH, 1)], axis=1)

    h1, a1 = _tc_project(_tc_layer1_body, x, W1, A1, 400)
    g1 = _make_sc_gat(HEADS, HIDDEN)(h1, a1.reshape(-1), rp_pad, src_pad, bias1)
    g1 = g1.reshape(N, HEADS * HIDDEN)

    h2, a2 = _tc_project(_tc_layer2_body, g1, W2, A2, 400)
    out = _make_sc_gat(1, OUT_CH)(h2, a2.reshape(-1), rp_pad, src_pad, bias2)
    return out.reshape(N, OUT_CH)
